# Initial kernel scaffold; baseline (speedup 1.0000x reference)
import jax
import jax.numpy as jnp
from jax.experimental import pallas as pl


def kernel(features, edge_index, W1, b1, W2, b2):
    raise NotImplementedError("write your pallas kernel here")


# speedup vs baseline: 18.4809x; 18.4809x over previous
import functools

import jax
import jax.numpy as jnp
from jax import lax
from jax.experimental import pallas as pl
from jax.experimental.pallas import tpu as pltpu
from jax.experimental.pallas import tpu_sc as plsc

N = 10000
E = 320000
D_IN = 128
HID = 16
NCLS = 32

NW = 32
NP = 10112
RPT = NP // 16
CH = 128
NCH = 80
EPW = CH * NCH
EP = NW * EPW

_mesh = plsc.VectorSubcoreMesh(core_axis_name="c", subcore_axis_name="s")
_sc_params = pltpu.CompilerParams(use_tc_tiling_on_sc=False)


@functools.partial(
    pl.kernel,
    out_type=(
        jax.ShapeDtypeStruct((2, NP), jnp.float32),
        jax.ShapeDtypeStruct((2, NP), jnp.float32),
    ),
    mesh=_mesh,
    scratch_types=[
        pltpu.VMEM((NCH, CH), jnp.int32),
        pltpu.VMEM((NCH, CH), jnp.int32),
        pltpu.VMEM((CH,), jnp.float32),
        pltpu.VMEM((RPT,), jnp.float32),
        pltpu.VMEM_SHARED((NP,), jnp.float32),
        pltpu.VMEM_SHARED((NP,), jnp.float32),
    ],
    compiler_params=_sc_params,
)
def _deg_kernel(src_hbm, dst_hbm, ones_hbm, z1_hbm, dego_hbm, degi_hbm,
                src_v, dst_v, ones_v, z1_v, dego_sh, degi_sh):
    c = lax.axis_index("c")
    s = lax.axis_index("s")
    wid = c * 16 + s
    pltpu.sync_copy(src_hbm.at[wid], src_v)
    pltpu.sync_copy(dst_hbm.at[wid], dst_v)
    pltpu.sync_copy(ones_hbm, ones_v)
    pltpu.sync_copy(z1_hbm, z1_v)
    sl = pl.ds(s * RPT, RPT)
    pltpu.sync_copy(z1_v, dego_sh.at[sl])
    pltpu.sync_copy(z1_v, degi_sh.at[sl])
    plsc.subcore_barrier()

    def body(j, carry):
        pltpu.sync_copy(ones_v, dego_sh.at[src_v.at[j]], add=True)
        pltpu.sync_copy(ones_v, degi_sh.at[dst_v.at[j]], add=True)
        return carry

    lax.fori_loop(0, NCH, body, None)
    plsc.subcore_barrier()
    pltpu.sync_copy(dego_sh.at[sl], dego_hbm.at[c].at[sl])
    pltpu.sync_copy(degi_sh.at[sl], degi_hbm.at[c].at[sl])


def _make_edge_kernel(F):
    @functools.partial(
        pl.kernel,
        out_type=jax.ShapeDtypeStruct((2, NP, F), jnp.float32),
        mesh=_mesh,
        scratch_types=[
            pltpu.VMEM((NCH, CH), jnp.int32),
            pltpu.VMEM((NCH, CH), jnp.int32),
            pltpu.VMEM((CH, F), jnp.float32),
            pltpu.VMEM((CH, F), jnp.float32),
            pltpu.VMEM((RPT, F), jnp.float32),
            pltpu.VMEM_SHARED((NP, F), jnp.float32),
            pltpu.SemaphoreType.DMA,
            pltpu.SemaphoreType.DMA,
        ],
        compiler_params=_sc_params,
    )
    def edge_kernel(table_hbm, src_hbm, dst_hbm, zeros_hbm, out_hbm,
                    src_v, dst_v, buf_a, buf_b, zero_v, agg_sh,
                    sem_a, sem_b):
        c = lax.axis_index("c")
        s = lax.axis_index("s")
        wid = c * 16 + s
        pltpu.sync_copy(src_hbm.at[wid], src_v)
        pltpu.sync_copy(dst_hbm.at[wid], dst_v)
        pltpu.sync_copy(zeros_hbm, zero_v)
        sl = pl.ds(s * RPT, RPT)
        pltpu.sync_copy(zero_v, agg_sh.at[sl])
        plsc.subcore_barrier()

        pltpu.async_copy(table_hbm.at[src_v.at[0]], buf_a, sem_a)

        def body(i, carry):
            g = 2 * i
            pltpu.async_copy(table_hbm.at[src_v.at[g + 1]], buf_b, sem_b)
            pltpu.make_async_copy(table_hbm.at[src_v.at[g]], buf_a,
                                  sem_a).wait()
            pltpu.sync_copy(buf_a, agg_sh.at[dst_v.at[g]], add=True)
            pltpu.async_copy(table_hbm.at[src_v.at[g + 2]], buf_a, sem_a)
            pltpu.make_async_copy(table_hbm.at[src_v.at[g + 1]], buf_b,
                                  sem_b).wait()
            pltpu.sync_copy(buf_b, agg_sh.at[dst_v.at[g + 1]], add=True)
            return carry

        lax.fori_loop(0, (NCH - 2) // 2, body, None)
        pltpu.async_copy(table_hbm.at[src_v.at[NCH - 1]], buf_b, sem_b)
        pltpu.make_async_copy(table_hbm.at[src_v.at[NCH - 2]], buf_a,
                              sem_a).wait()
        pltpu.sync_copy(buf_a, agg_sh.at[dst_v.at[NCH - 2]], add=True)
        pltpu.make_async_copy(table_hbm.at[src_v.at[NCH - 1]], buf_b,
                              sem_b).wait()
        pltpu.sync_copy(buf_b, agg_sh.at[dst_v.at[NCH - 1]], add=True)

        plsc.subcore_barrier()
        pltpu.sync_copy(agg_sh.at[sl], out_hbm.at[c].at[sl])

    return edge_kernel


_edge16 = _make_edge_kernel(HID)
_edge32 = _make_edge_kernel(NCLS)


def _mm1_body(feat_ref, w1_ref, xw_ref):
    xw_ref[...] = jnp.dot(feat_ref[...], w1_ref[...],
                          preferred_element_type=jnp.float32)


def _norm_body(xw_ref, dpo_ref, dpi_ref, t1_ref, ns_ref, nd_ref):
    ns = lax.rsqrt(jnp.maximum(dpo_ref[0] + dpo_ref[1], 1.0))
    nd = lax.rsqrt(jnp.maximum(dpi_ref[0] + dpi_ref[1], 1.0))
    t1_ref[...] = xw_ref[...] * ns
    ns_ref[...] = ns
    nd_ref[...] = nd


def _mid_body(aggp_ref, nd_ref, ns_ref, b1_ref, w2_ref, t2_ref):
    h = (aggp_ref[0] + aggp_ref[1]) * nd_ref[...] + b1_ref[...]
    x = jnp.maximum(h, 0.0)
    t2_ref[...] = jnp.dot(x, w2_ref[...],
                          preferred_element_type=jnp.float32) * ns_ref[...]


def _out_body(aggp_ref, nd_ref, b2_ref, out_ref):
    out_ref[...] = (aggp_ref[0] + aggp_ref[1]) * nd_ref[...] + b2_ref[...]


def _tc_call(body, out_shapes, *args):
    return pl.pallas_call(body, out_shape=out_shapes)(*args)


def kernel(features, edge_index, W1, b1, W2, b2):
    f32 = jnp.float32
    pad_idx = jnp.full((EP - E,), NP - 1, jnp.int32)
    src_r = jnp.concatenate([edge_index[0], pad_idx]).reshape(NW, NCH, CH)
    dst_r = jnp.concatenate([edge_index[1], pad_idx]).reshape(NW, NCH, CH)
    ones1 = jnp.ones((CH,), f32)
    z1 = jnp.zeros((RPT,), f32)
    z16 = jnp.zeros((RPT, HID), f32)
    z32 = jnp.zeros((RPT, NCLS), f32)
    feat_pad = jnp.pad(features, ((0, NP - N), (0, 0)))

    dego_p, degi_p = _deg_kernel(src_r, dst_r, ones1, z1)
    xw1 = _tc_call(_mm1_body, jax.ShapeDtypeStruct((NP, HID), f32),
                   feat_pad, W1)

    t1, ns, nd = _tc_call(
        _norm_body,
        (jax.ShapeDtypeStruct((NP, HID), f32),
         jax.ShapeDtypeStruct((NP, 1), f32),
         jax.ShapeDtypeStruct((NP, 1), f32)),
        xw1, dego_p.reshape(2, NP, 1), degi_p.reshape(2, NP, 1))

    agg1_p = _edge16(t1, src_r, dst_r, z16)

    t2 = _tc_call(
        _mid_body,
        jax.ShapeDtypeStruct((NP, NCLS), f32),
        agg1_p, nd, ns, b1.reshape(1, HID), W2)

    agg2_p = _edge32(t2, src_r, dst_r, z32)

    out = _tc_call(
        _out_body,
        jax.ShapeDtypeStruct((NP, NCLS), f32),
        agg2_p, nd, b2.reshape(1, NCLS))

    return out[:N]


# speedup vs baseline: 19.2450x; 1.0413x over previous
import functools

import jax
import jax.numpy as jnp
from jax import lax
from jax.experimental import pallas as pl
from jax.experimental.pallas import tpu as pltpu
from jax.experimental.pallas import tpu_sc as plsc

N = 10000
E = 320000
D_IN = 128
HID = 16
NCLS = 32

NW = 32
NP = 10112
RPT = NP // 16
CH = 128
NCH = 80
EPW = CH * NCH
EP = NW * EPW

_mesh = plsc.VectorSubcoreMesh(core_axis_name="c", subcore_axis_name="s")
_sc_params = pltpu.CompilerParams(use_tc_tiling_on_sc=False)


@functools.partial(
    pl.kernel,
    out_type=(
        jax.ShapeDtypeStruct((2, NP), jnp.float32),
        jax.ShapeDtypeStruct((2, NP), jnp.float32),
    ),
    mesh=_mesh,
    scratch_types=[
        pltpu.VMEM((NCH, CH), jnp.int32),
        pltpu.VMEM((NCH, CH), jnp.int32),
        pltpu.VMEM((CH,), jnp.float32),
        pltpu.VMEM((RPT,), jnp.float32),
        pltpu.VMEM_SHARED((NP,), jnp.float32),
        pltpu.VMEM_SHARED((NP,), jnp.float32),
        pltpu.SemaphoreType.DMA,
        pltpu.SemaphoreType.DMA,
    ],
    compiler_params=_sc_params,
)
def _deg_kernel(src_hbm, dst_hbm, ones_hbm, z1_hbm, dego_hbm, degi_hbm,
                src_v, dst_v, ones_v, z1_v, dego_sh, degi_sh, sem_a, sem_b):
    c = lax.axis_index("c")
    s = lax.axis_index("s")
    wid = c * 16 + s
    pltpu.sync_copy(src_hbm.at[wid], src_v)
    pltpu.sync_copy(dst_hbm.at[wid], dst_v)
    pltpu.sync_copy(ones_hbm, ones_v)
    pltpu.sync_copy(z1_hbm, z1_v)
    sl = pl.ds(s * RPT, RPT)
    pltpu.sync_copy(z1_v, dego_sh.at[sl])
    pltpu.sync_copy(z1_v, degi_sh.at[sl])
    plsc.subcore_barrier()

    pltpu.async_copy(ones_v, dego_sh.at[src_v.at[0]], sem_a, add=True)
    pltpu.async_copy(ones_v, degi_sh.at[dst_v.at[0]], sem_b, add=True)

    def body(j, carry):
        pltpu.async_copy(ones_v, dego_sh.at[src_v.at[j]], sem_a, add=True)
        pltpu.async_copy(ones_v, degi_sh.at[dst_v.at[j]], sem_b, add=True)
        pltpu.make_async_copy(ones_v, dego_sh.at[src_v.at[j - 1]],
                              sem_a).wait()
        pltpu.make_async_copy(ones_v, degi_sh.at[dst_v.at[j - 1]],
                              sem_b).wait()
        return carry

    lax.fori_loop(1, NCH, body, None)
    pltpu.make_async_copy(ones_v, dego_sh.at[src_v.at[NCH - 1]],
                          sem_a).wait()
    pltpu.make_async_copy(ones_v, degi_sh.at[dst_v.at[NCH - 1]],
                          sem_b).wait()
    plsc.subcore_barrier()
    pltpu.sync_copy(dego_sh.at[sl], dego_hbm.at[c].at[sl])
    pltpu.sync_copy(degi_sh.at[sl], degi_hbm.at[c].at[sl])


def _make_edge_kernel(F):
    @functools.partial(
        pl.kernel,
        out_type=jax.ShapeDtypeStruct((2, NP, F), jnp.float32),
        mesh=_mesh,
        scratch_types=[
            pltpu.VMEM((NCH, CH), jnp.int32),
            pltpu.VMEM((NCH, CH), jnp.int32),
            pltpu.VMEM((CH, F), jnp.float32),
            pltpu.VMEM((CH, F), jnp.float32),
            pltpu.VMEM((CH, F), jnp.float32),
            pltpu.VMEM((CH, F), jnp.float32),
            pltpu.VMEM((RPT, F), jnp.float32),
            pltpu.VMEM_SHARED((NP, F), jnp.float32),
            pltpu.SemaphoreType.DMA,
            pltpu.SemaphoreType.DMA,
            pltpu.SemaphoreType.DMA,
            pltpu.SemaphoreType.DMA,
            pltpu.SemaphoreType.DMA,
            pltpu.SemaphoreType.DMA,
            pltpu.SemaphoreType.DMA,
            pltpu.SemaphoreType.DMA,
        ],
        compiler_params=_sc_params,
    )
    def edge_kernel(table_hbm, src_hbm, dst_hbm, zeros_hbm, out_hbm,
                    src_v, dst_v, b0, b1, b2, b3, zero_v, agg_sh,
                    g0, g1, g2, g3, s0, s1, s2, s3):
        c = lax.axis_index("c")
        s = lax.axis_index("s")
        wid = c * 16 + s
        pltpu.sync_copy(src_hbm.at[wid], src_v)
        pltpu.sync_copy(dst_hbm.at[wid], dst_v)
        pltpu.sync_copy(zeros_hbm, zero_v)
        sl = pl.ds(s * RPT, RPT)
        pltpu.sync_copy(zero_v, agg_sh.at[sl])
        plsc.subcore_barrier()

        bufs = (b0, b1, b2, b3)
        gsems = (g0, g1, g2, g3)
        ssems = (s0, s1, s2, s3)

        def gat(j, b):
            pltpu.async_copy(table_hbm.at[src_v.at[j]], bufs[b], gsems[b])

        def gwait(j, b):
            pltpu.make_async_copy(table_hbm.at[src_v.at[j]], bufs[b],
                                  gsems[b]).wait()

        def sct(j, b):
            pltpu.async_copy(bufs[b], agg_sh.at[dst_v.at[j]], ssems[b],
                             add=True)

        def swait(j, b):
            pltpu.make_async_copy(bufs[b], agg_sh.at[dst_v.at[j]],
                                  ssems[b]).wait()

        gat(0, 0)
        gat(1, 1)
        gat(2, 2)
        gwait(0, 0)
        sct(0, 0)
        gat(3, 3)
        gwait(1, 1)
        sct(1, 1)
        swait(0, 0)
        gat(4, 0)
        gwait(2, 2)
        sct(2, 2)
        swait(1, 1)
        gat(5, 1)
        gwait(3, 3)
        sct(3, 3)

        def body(i, carry):
            for b in range(4):
                j = 4 * i + b
                bb = (b + 2) % 4
                swait(j - 2, bb)
                gat(j + 2, bb)
                gwait(j, b)
                sct(j, b)
            return carry

        lax.fori_loop(1, (NCH - 8) // 4 + 1, body, None)
        swait(NCH - 6, 2)
        gat(NCH - 2, 2)
        gwait(NCH - 4, 0)
        sct(NCH - 4, 0)
        swait(NCH - 5, 3)
        gat(NCH - 1, 3)
        gwait(NCH - 3, 1)
        sct(NCH - 3, 1)
        gwait(NCH - 2, 2)
        sct(NCH - 2, 2)
        gwait(NCH - 1, 3)
        sct(NCH - 1, 3)
        swait(NCH - 4, 0)
        swait(NCH - 3, 1)
        swait(NCH - 2, 2)
        swait(NCH - 1, 3)

        plsc.subcore_barrier()
        pltpu.sync_copy(agg_sh.at[sl], out_hbm.at[c].at[sl])

    return edge_kernel


_edge16 = _make_edge_kernel(HID)
_edge32 = _make_edge_kernel(NCLS)


def _mm1_body(feat_ref, w1_ref, xw_ref):
    xw_ref[...] = jnp.dot(feat_ref[...], w1_ref[...],
                          preferred_element_type=jnp.float32)


def _norm_body(xw_ref, dpo_ref, dpi_ref, t1_ref, ns_ref, nd_ref):
    ns = lax.rsqrt(jnp.maximum(dpo_ref[0] + dpo_ref[1], 1.0))
    nd = lax.rsqrt(jnp.maximum(dpi_ref[0] + dpi_ref[1], 1.0))
    t1_ref[...] = xw_ref[...] * ns
    ns_ref[...] = ns
    nd_ref[...] = nd


def _mid_body(aggp_ref, nd_ref, ns_ref, b1_ref, w2_ref, t2_ref):
    h = (aggp_ref[0] + aggp_ref[1]) * nd_ref[...] + b1_ref[...]
    x = jnp.maximum(h, 0.0)
    t2_ref[...] = jnp.dot(x, w2_ref[...],
                          preferred_element_type=jnp.float32) * ns_ref[...]


def _out_body(aggp_ref, nd_ref, b2_ref, out_ref):
    out_ref[...] = (aggp_ref[0] + aggp_ref[1]) * nd_ref[...] + b2_ref[...]


def _tc_call(body, out_shapes, *args):
    return pl.pallas_call(body, out_shape=out_shapes)(*args)


def kernel(features, edge_index, W1, b1, W2, b2):
    f32 = jnp.float32
    pad_idx = jnp.full((EP - E,), NP - 1, jnp.int32)
    src_r = jnp.concatenate([edge_index[0], pad_idx]).reshape(NW, NCH, CH)
    dst_r = jnp.concatenate([edge_index[1], pad_idx]).reshape(NW, NCH, CH)
    ones1 = jnp.ones((CH,), f32)
    z1 = jnp.zeros((RPT,), f32)
    z16 = jnp.zeros((RPT, HID), f32)
    z32 = jnp.zeros((RPT, NCLS), f32)
    feat_pad = jnp.pad(features, ((0, NP - N), (0, 0)))

    dego_p, degi_p = _deg_kernel(src_r, dst_r, ones1, z1)
    xw1 = _tc_call(_mm1_body, jax.ShapeDtypeStruct((NP, HID), f32),
                   feat_pad, W1)

    t1, ns, nd = _tc_call(
        _norm_body,
        (jax.ShapeDtypeStruct((NP, HID), f32),
         jax.ShapeDtypeStruct((NP, 1), f32),
         jax.ShapeDtypeStruct((NP, 1), f32)),
        xw1, dego_p.reshape(2, NP, 1), degi_p.reshape(2, NP, 1))

    agg1_p = _edge16(t1, src_r, dst_r, z16)

    t2 = _tc_call(
        _mid_body,
        jax.ShapeDtypeStruct((NP, NCLS), f32),
        agg1_p, nd, ns, b1.reshape(1, HID), W2)

    agg2_p = _edge32(t2, src_r, dst_r, z32)

    out = _tc_call(
        _out_body,
        jax.ShapeDtypeStruct((NP, NCLS), f32),
        agg2_p, nd, b2.reshape(1, NCLS))

    return out[:N]


# speedup vs baseline: 28.2279x; 1.4668x over previous
import functools

import jax
import jax.numpy as jnp
from jax import lax
from jax.experimental import pallas as pl
from jax.experimental.pallas import tpu as pltpu
from jax.experimental.pallas import tpu_sc as plsc

N = 10000
E = 320000
D_IN = 128
HID = 16
NCLS = 32

NW = 32
NP = 10112
RPT = NP // 16
CH = 128
NCH = 80
EPW = CH * NCH
EP = NW * EPW

_mesh = plsc.VectorSubcoreMesh(core_axis_name="c", subcore_axis_name="s")
_sc_params = pltpu.CompilerParams(use_tc_tiling_on_sc=False)


@functools.partial(
    pl.kernel,
    out_type=(
        jax.ShapeDtypeStruct((2, NP), jnp.float32),
        jax.ShapeDtypeStruct((2, NP), jnp.float32),
    ),
    mesh=_mesh,
    scratch_types=[
        pltpu.VMEM((NCH, CH), jnp.int32),
        pltpu.VMEM((NCH, CH), jnp.int32),
        pltpu.VMEM((CH,), jnp.float32),
        pltpu.VMEM((RPT,), jnp.float32),
        pltpu.VMEM_SHARED((NP,), jnp.float32),
        pltpu.VMEM_SHARED((NP,), jnp.float32),
        pltpu.SemaphoreType.DMA,
        pltpu.SemaphoreType.DMA,
    ],
    compiler_params=_sc_params,
)
def _deg_kernel(src_hbm, dst_hbm, ones_hbm, z1_hbm, dego_hbm, degi_hbm,
                src_v, dst_v, ones_v, z1_v, dego_sh, degi_sh, sem_a, sem_b):
    c = lax.axis_index("c")
    s = lax.axis_index("s")
    wid = c * 16 + s
    pltpu.sync_copy(src_hbm.at[wid], src_v)
    pltpu.sync_copy(dst_hbm.at[wid], dst_v)
    pltpu.sync_copy(ones_hbm, ones_v)
    pltpu.sync_copy(z1_hbm, z1_v)
    sl = pl.ds(s * RPT, RPT)
    pltpu.sync_copy(z1_v, dego_sh.at[sl])
    pltpu.sync_copy(z1_v, degi_sh.at[sl])
    plsc.subcore_barrier()

    pltpu.async_copy(ones_v, dego_sh.at[src_v.at[0]], sem_a, add=True)
    pltpu.async_copy(ones_v, degi_sh.at[dst_v.at[0]], sem_b, add=True)

    def body(j, carry):
        pltpu.async_copy(ones_v, dego_sh.at[src_v.at[j]], sem_a, add=True)
        pltpu.async_copy(ones_v, degi_sh.at[dst_v.at[j]], sem_b, add=True)
        pltpu.make_async_copy(ones_v, dego_sh.at[src_v.at[j - 1]],
                              sem_a).wait()
        pltpu.make_async_copy(ones_v, degi_sh.at[dst_v.at[j - 1]],
                              sem_b).wait()
        return carry

    lax.fori_loop(1, NCH, body, None)
    pltpu.make_async_copy(ones_v, dego_sh.at[src_v.at[NCH - 1]],
                          sem_a).wait()
    pltpu.make_async_copy(ones_v, degi_sh.at[dst_v.at[NCH - 1]],
                          sem_b).wait()
    plsc.subcore_barrier()
    pltpu.sync_copy(dego_sh.at[sl], dego_hbm.at[c].at[sl])
    pltpu.sync_copy(degi_sh.at[sl], degi_hbm.at[c].at[sl])


def _make_edge_kernel(F):
    @functools.partial(
        pl.kernel,
        out_type=jax.ShapeDtypeStruct((2, NP, F), jnp.float32),
        mesh=_mesh,
        scratch_types=[
            pltpu.VMEM((NCH, CH), jnp.int32),
            pltpu.VMEM((NCH, CH), jnp.int32),
            pltpu.VMEM((CH, F), jnp.float32),
            pltpu.VMEM((CH, F), jnp.float32),
            pltpu.VMEM((CH, F), jnp.float32),
            pltpu.VMEM((CH, F), jnp.float32),
            pltpu.VMEM((RPT, F), jnp.float32),
            pltpu.VMEM_SHARED((NP, F), jnp.float32),
            pltpu.VMEM_SHARED((NP, F), jnp.float32),
            pltpu.SemaphoreType.DMA,
            pltpu.SemaphoreType.DMA,
            pltpu.SemaphoreType.DMA,
            pltpu.SemaphoreType.DMA,
            pltpu.SemaphoreType.DMA,
            pltpu.SemaphoreType.DMA,
            pltpu.SemaphoreType.DMA,
            pltpu.SemaphoreType.DMA,
        ],
        compiler_params=_sc_params,
    )
    def edge_kernel(table_hbm, src_hbm, dst_hbm, zeros_hbm, out_hbm,
                    src_v, dst_v, b0, b1, b2, b3, zero_v, agg_sh, tab_sh,
                    g0, g1, g2, g3, s0, s1, s2, s3):
        c = lax.axis_index("c")
        s = lax.axis_index("s")
        wid = c * 16 + s
        pltpu.sync_copy(src_hbm.at[wid], src_v)
        pltpu.sync_copy(dst_hbm.at[wid], dst_v)
        pltpu.sync_copy(zeros_hbm, zero_v)
        sl = pl.ds(s * RPT, RPT)
        pltpu.sync_copy(table_hbm.at[sl], tab_sh.at[sl])
        pltpu.sync_copy(zero_v, agg_sh.at[sl])
        plsc.subcore_barrier()

        bufs = (b0, b1, b2, b3)
        gsems = (g0, g1, g2, g3)
        ssems = (s0, s1, s2, s3)

        def gat(j, b):
            pltpu.async_copy(tab_sh.at[src_v.at[j]], bufs[b], gsems[b])

        def gwait(j, b):
            pltpu.make_async_copy(tab_sh.at[src_v.at[j]], bufs[b],
                                  gsems[b]).wait()

        def sct(j, b):
            pltpu.async_copy(bufs[b], agg_sh.at[dst_v.at[j]], ssems[b],
                             add=True)

        def swait(j, b):
            pltpu.make_async_copy(bufs[b], agg_sh.at[dst_v.at[j]],
                                  ssems[b]).wait()

        gat(0, 0)
        gat(1, 1)
        gat(2, 2)
        gwait(0, 0)
        sct(0, 0)
        gat(3, 3)
        gwait(1, 1)
        sct(1, 1)
        swait(0, 0)
        gat(4, 0)
        gwait(2, 2)
        sct(2, 2)
        swait(1, 1)
        gat(5, 1)
        gwait(3, 3)
        sct(3, 3)

        def body(i, carry):
            for b in range(4):
                j = 4 * i + b
                bb = (b + 2) % 4
                swait(j - 2, bb)
                gat(j + 2, bb)
                gwait(j, b)
                sct(j, b)
            return carry

        lax.fori_loop(1, (NCH - 8) // 4 + 1, body, None)
        swait(NCH - 6, 2)
        gat(NCH - 2, 2)
        gwait(NCH - 4, 0)
        sct(NCH - 4, 0)
        swait(NCH - 5, 3)
        gat(NCH - 1, 3)
        gwait(NCH - 3, 1)
        sct(NCH - 3, 1)
        gwait(NCH - 2, 2)
        sct(NCH - 2, 2)
        gwait(NCH - 1, 3)
        sct(NCH - 1, 3)
        swait(NCH - 4, 0)
        swait(NCH - 3, 1)
        swait(NCH - 2, 2)
        swait(NCH - 1, 3)

        plsc.subcore_barrier()
        pltpu.sync_copy(agg_sh.at[sl], out_hbm.at[c].at[sl])

    return edge_kernel


_edge16 = _make_edge_kernel(HID)
_edge32 = _make_edge_kernel(NCLS)


def _mm1_body(feat_ref, w1_ref, xw_ref):
    xw_ref[...] = jnp.dot(feat_ref[...], w1_ref[...],
                          preferred_element_type=jnp.float32)


def _norm_body(xw_ref, dpo_ref, dpi_ref, t1_ref, ns_ref, nd_ref):
    ns = lax.rsqrt(jnp.maximum(dpo_ref[0] + dpo_ref[1], 1.0))
    nd = lax.rsqrt(jnp.maximum(dpi_ref[0] + dpi_ref[1], 1.0))
    t1_ref[...] = xw_ref[...] * ns
    ns_ref[...] = ns
    nd_ref[...] = nd


def _mid_body(aggp_ref, nd_ref, ns_ref, b1_ref, w2_ref, t2_ref):
    h = (aggp_ref[0] + aggp_ref[1]) * nd_ref[...] + b1_ref[...]
    x = jnp.maximum(h, 0.0)
    t2_ref[...] = jnp.dot(x, w2_ref[...],
                          preferred_element_type=jnp.float32) * ns_ref[...]


def _out_body(aggp_ref, nd_ref, b2_ref, out_ref):
    out_ref[...] = (aggp_ref[0] + aggp_ref[1]) * nd_ref[...] + b2_ref[...]


def _tc_call(body, out_shapes, *args):
    return pl.pallas_call(body, out_shape=out_shapes)(*args)


def kernel(features, edge_index, W1, b1, W2, b2):
    f32 = jnp.float32
    pad_idx = jnp.full((EP - E,), NP - 1, jnp.int32)
    src_r = jnp.concatenate([edge_index[0], pad_idx]).reshape(NW, NCH, CH)
    dst_r = jnp.concatenate([edge_index[1], pad_idx]).reshape(NW, NCH, CH)
    ones1 = jnp.ones((CH,), f32)
    z1 = jnp.zeros((RPT,), f32)
    z16 = jnp.zeros((RPT, HID), f32)
    z32 = jnp.zeros((RPT, NCLS), f32)
    feat_pad = jnp.pad(features, ((0, NP - N), (0, 0)))

    dego_p, degi_p = _deg_kernel(src_r, dst_r, ones1, z1)
    xw1 = _tc_call(_mm1_body, jax.ShapeDtypeStruct((NP, HID), f32),
                   feat_pad, W1)

    t1, ns, nd = _tc_call(
        _norm_body,
        (jax.ShapeDtypeStruct((NP, HID), f32),
         jax.ShapeDtypeStruct((NP, 1), f32),
         jax.ShapeDtypeStruct((NP, 1), f32)),
        xw1, dego_p.reshape(2, NP, 1), degi_p.reshape(2, NP, 1))

    agg1_p = _edge16(t1, src_r, dst_r, z16)

    t2 = _tc_call(
        _mid_body,
        jax.ShapeDtypeStruct((NP, NCLS), f32),
        agg1_p, nd, ns, b1.reshape(1, HID), W2)

    agg2_p = _edge32(t2, src_r, dst_r, z32)

    out = _tc_call(
        _out_body,
        jax.ShapeDtypeStruct((NP, NCLS), f32),
        agg2_p, nd, b2.reshape(1, NCLS))

    return out[:N]
